# baseline (speedup 1.0000x reference)
import functools

import jax
import jax.numpy as jnp
from jax import lax
from jax.experimental import pallas as pl
from jax.experimental.pallas import tpu as pltpu
from jax.experimental.pallas import tpu_sc as plsc

M = 8
L = 8192
NW = 32
NCHUNK = 2
CPW = 1024 // NCHUNK // NW
NSEG = 32
VPS = 16
BIGF = -3.0
BIGI = 2 * L


def _lane_reduce(v, op):
    acc = v[0]
    for i in range(1, 16):
        acc = op(acc, v[i])
    return acc


def _sc_select(a_hbm, out_hbm, slab_v, acc_v, res_v, sem):
    wid = lax.axis_index("s") * 2 + lax.axis_index("c")
    lane = lax.iota(jnp.int32, 16)

    def per_column(cl, _):
        col = wid * CPW + cl
        pltpu.sync_copy(a_hbm.at[col], slab_v)

        def seg_init(s, _):
            def seg_scan(t, acc):
                return jnp.maximum(acc, slab_v[pl.ds((s * VPS + t) * 16, 16)])

            acc = lax.fori_loop(0, VPS, seg_scan, jnp.full((16,), -1.0, jnp.float32))
            acc_v[pl.ds(s * 16, 16)] = acc
            return _

        lax.fori_loop(0, NSEG, seg_init, 0)

        def per_round(m, p_acc):
            def gmax_scan(s, g):
                return jnp.maximum(g, acc_v[pl.ds(s * 16, 16)])

            g = lax.fori_loop(0, NSEG, gmax_scan, jnp.full((16,), -2.0, jnp.float32))
            gv_s = _lane_reduce(g, jnp.maximum)
            gv = jnp.zeros((16,), jnp.float32) + gv_s

            def seg_find(s, best):
                accs = acc_v[pl.ds(s * 16, 16)]
                cand = jnp.where(accs == gv, s, BIGI)
                return jnp.minimum(best, cand)

            s_vec = lax.fori_loop(
                0, NSEG, seg_find, jnp.full((16,), BIGI, jnp.int32)
            )
            s_star = _lane_reduce(s_vec, jnp.minimum)

            def row_find(t, best):
                vreg = slab_v[pl.ds((s_star * VPS + t) * 16, 16)]
                eidx = (s_star * VPS + t) * 16 + lane
                cand = jnp.where(vreg == gv, eidx, BIGI)
                return jnp.minimum(best, cand)

            k_vec = lax.fori_loop(
                0, VPS, row_find, jnp.full((16,), BIGI, jnp.int32)
            )
            k = _lane_reduce(k_vec, jnp.minimum)

            p_acc = jnp.where(lane == m, (L + k) // (k + 1), p_acc)

            plsc.store_scatter(
                slab_v,
                [jnp.full((16,), 0, jnp.int32) + k],
                jnp.full((16,), BIGF, jnp.float32),
                mask=lane == 0,
            )

            def seg_rescan(t, acc):
                return jnp.maximum(acc, slab_v[pl.ds((s_star * VPS + t) * 16, 16)])

            newacc = lax.fori_loop(0, VPS, seg_rescan, jnp.full((16,), -1.0, jnp.float32))
            acc_v[pl.ds(s_star * 16, 16)] = newacc
            return p_acc

        p_acc = lax.fori_loop(0, M, per_round, jnp.zeros((16,), jnp.int32))
        res_v[pl.ds(cl * 16, 16)] = p_acc
        return _

    lax.fori_loop(0, CPW, per_column, 0)
    pltpu.sync_copy(res_v, out_hbm.at[pl.ds(wid * (CPW * 16), CPW * 16)])


def kernel(x_input):
    b, length, d = x_input.shape
    bc = b // NCHUNK
    mesh = plsc.VectorSubcoreMesh(core_axis_name="c", subcore_axis_name="s")
    sc = functools.partial(
        pl.kernel,
        mesh=mesh,
        compiler_params=pltpu.CompilerParams(needs_layout_passes=False),
        out_type=jax.ShapeDtypeStruct((bc * d * 16,), jnp.int32),
        scratch_types=[
            pltpu.VMEM((length,), jnp.float32),
            pltpu.VMEM((NSEG * 16,), jnp.float32),
            pltpu.VMEM((CPW * 16,), jnp.int32),
            pltpu.SemaphoreType.DMA,
        ],
    )(_sc_select)
    flats = []
    for i in range(NCHUNK):
        xc = x_input[i * bc : (i + 1) * bc]
        ac = jnp.abs(jnp.fft.fft(xc, axis=1))
        a_t = jnp.transpose(ac, (0, 2, 1)).reshape(bc * d, length)
        flats.append(sc(a_t))
    flat = jnp.concatenate(flats)
    p = jnp.transpose(flat.reshape(b, d, 16)[:, :, :M], (0, 2, 1))
    return p.astype(jnp.int64)


# speedup vs baseline: 1.0954x; 1.0954x over previous
import functools

import jax
import jax.numpy as jnp
from jax import lax
from jax.experimental import pallas as pl
from jax.experimental.pallas import tpu as pltpu
from jax.experimental.pallas import tpu_sc as plsc

M = 8
L = 8192
NW = 32
CPW = 1024 // NW
NSEG = 32
VPS = 16
BIGF = -3.0
BIGI = 2 * L


_GDN = lax.GatherDimensionNumbers(
    offset_dims=(), collapsed_slice_dims=(0,), start_index_map=(0,)
)


def _lane_all(v, op, lane):
    for sh in (8, 4, 2, 1):
        perm = lane ^ sh
        sh_v = lax.gather(
            v, perm[:, None], _GDN, (1,),
            mode=lax.GatherScatterMode.PROMISE_IN_BOUNDS,
        )
        v = op(v, sh_v)
    return v


def _sc_select(a_hbm, out_hbm, slab_v, acc_v, res_v, sem):
    wid = lax.axis_index("s") * 2 + lax.axis_index("c")
    lane = lax.iota(jnp.int32, 16)

    def per_column(cl, _):
        col = wid * CPW + cl
        pltpu.sync_copy(a_hbm.at[col], slab_v)

        def seg_init(s, _):
            def seg_scan(t, acc):
                return jnp.maximum(acc, slab_v[pl.ds((s * VPS + t) * 16, 16)])

            acc = lax.fori_loop(0, VPS, seg_scan, jnp.full((16,), -1.0, jnp.float32))
            acc_v[pl.ds(s * 16, 16)] = acc
            return _

        lax.fori_loop(0, NSEG, seg_init, 0)

        def per_round(m, p_acc):
            def gmax_scan(s, g):
                return jnp.maximum(g, acc_v[pl.ds(s * 16, 16)])

            g = lax.fori_loop(0, NSEG, gmax_scan, jnp.full((16,), -2.0, jnp.float32))
            gv = _lane_all(g, jnp.maximum, lane)

            def seg_find(s, best):
                accs = acc_v[pl.ds(s * 16, 16)]
                cand = jnp.where(accs == gv, s, BIGI)
                return jnp.minimum(best, cand)

            s_vec = lax.fori_loop(
                0, NSEG, seg_find, jnp.full((16,), BIGI, jnp.int32)
            )
            s_star = _lane_all(s_vec, jnp.minimum, lane)[0]

            def row_find(t, best):
                vreg = slab_v[pl.ds((s_star * VPS + t) * 16, 16)]
                eidx = (s_star * VPS + t) * 16 + lane
                cand = jnp.where(vreg == gv, eidx, BIGI)
                return jnp.minimum(best, cand)

            k_vec = lax.fori_loop(
                0, VPS, row_find, jnp.full((16,), BIGI, jnp.int32)
            )
            k = _lane_all(k_vec, jnp.minimum, lane)[0]

            p_acc = jnp.where(lane == m, (L + k) // (k + 1), p_acc)

            plsc.store_scatter(
                slab_v,
                [jnp.full((16,), 0, jnp.int32) + k],
                jnp.full((16,), BIGF, jnp.float32),
                mask=lane == 0,
            )

            def seg_rescan(t, acc):
                return jnp.maximum(acc, slab_v[pl.ds((s_star * VPS + t) * 16, 16)])

            newacc = lax.fori_loop(0, VPS, seg_rescan, jnp.full((16,), -1.0, jnp.float32))
            acc_v[pl.ds(s_star * 16, 16)] = newacc
            return p_acc

        p_acc = lax.fori_loop(0, M, per_round, jnp.zeros((16,), jnp.int32))
        res_v[pl.ds(cl * 16, 16)] = p_acc
        return _

    lax.fori_loop(0, CPW, per_column, 0)
    pltpu.sync_copy(res_v, out_hbm.at[pl.ds(wid * (CPW * 16), CPW * 16)])


def kernel(x_input):
    b, length, d = x_input.shape
    x_DFT = jnp.fft.fft(x_input, axis=1)
    a = jnp.abs(x_DFT)
    a_t = jnp.transpose(a, (0, 2, 1)).reshape(b * d, length)
    mesh = plsc.VectorSubcoreMesh(core_axis_name="c", subcore_axis_name="s")
    sc = functools.partial(
        pl.kernel,
        mesh=mesh,
        compiler_params=pltpu.CompilerParams(needs_layout_passes=False),
        out_type=jax.ShapeDtypeStruct((b * d * 16,), jnp.int32),
        scratch_types=[
            pltpu.VMEM((length,), jnp.float32),
            pltpu.VMEM((NSEG * 16,), jnp.float32),
            pltpu.VMEM((CPW * 16,), jnp.int32),
            pltpu.SemaphoreType.DMA,
        ],
    )(_sc_select)
    flat = sc(a_t)
    p = jnp.transpose(flat.reshape(b, d, 16)[:, :, :M], (0, 2, 1))
    return p.astype(jnp.int64)


# speedup vs baseline: 1.1164x; 1.0192x over previous
import functools

import jax
import jax.numpy as jnp
from jax import lax
from jax.experimental import pallas as pl
from jax.experimental.pallas import tpu as pltpu
from jax.experimental.pallas import tpu_sc as plsc

M = 8
L = 8192
NW = 32
CPW = 1024 // NW
NSEG = 32
VPS = 16
BIGF = -3.0
BIGI = 2 * L


_GDN = lax.GatherDimensionNumbers(
    offset_dims=(), collapsed_slice_dims=(0,), start_index_map=(0,)
)


def _lane_all(v, op, lane):
    for sh in (8, 4, 2, 1):
        perm = lane ^ sh
        sh_v = lax.gather(
            v, perm[:, None], _GDN, (1,),
            mode=lax.GatherScatterMode.PROMISE_IN_BOUNDS,
        )
        v = op(v, sh_v)
    return v


def _sc_select(a_hbm, out_hbm, slab_a, slab_b, acc_v, res_v, sem_a, sem_b):
    wid = lax.axis_index("s") * 2 + lax.axis_index("c")
    lane = lax.iota(jnp.int32, 16)

    def per_column(cl, slab_v):

        def seg_init(s, _):
            def seg_scan(t, acc):
                return jnp.maximum(acc, slab_v[pl.ds((s * VPS + t) * 16, 16)])

            acc = lax.fori_loop(0, VPS, seg_scan, jnp.full((16,), -1.0, jnp.float32))
            acc_v[pl.ds(s * 16, 16)] = acc
            return _

        lax.fori_loop(0, NSEG, seg_init, 0)

        def per_round(m, p_acc):
            def gmax_scan(s, g):
                return jnp.maximum(g, acc_v[pl.ds(s * 16, 16)])

            g = lax.fori_loop(0, NSEG, gmax_scan, jnp.full((16,), -2.0, jnp.float32))
            gv = _lane_all(g, jnp.maximum, lane)

            def seg_find(s, best):
                accs = acc_v[pl.ds(s * 16, 16)]
                cand = jnp.where(accs == gv, s, BIGI)
                return jnp.minimum(best, cand)

            s_vec = lax.fori_loop(
                0, NSEG, seg_find, jnp.full((16,), BIGI, jnp.int32)
            )
            s_star = _lane_all(s_vec, jnp.minimum, lane)[0]

            def row_find(t, best):
                vreg = slab_v[pl.ds((s_star * VPS + t) * 16, 16)]
                eidx = (s_star * VPS + t) * 16 + lane
                cand = jnp.where(vreg == gv, eidx, BIGI)
                return jnp.minimum(best, cand)

            k_vec = lax.fori_loop(
                0, VPS, row_find, jnp.full((16,), BIGI, jnp.int32)
            )
            k = _lane_all(k_vec, jnp.minimum, lane)[0]

            p_acc = jnp.where(lane == m, (L + k) // (k + 1), p_acc)

            plsc.store_scatter(
                slab_v,
                [jnp.full((16,), 0, jnp.int32) + k],
                jnp.full((16,), BIGF, jnp.float32),
                mask=lane == 0,
            )

            def seg_rescan(t, acc):
                return jnp.maximum(acc, slab_v[pl.ds((s_star * VPS + t) * 16, 16)])

            newacc = lax.fori_loop(0, VPS, seg_rescan, jnp.full((16,), -1.0, jnp.float32))
            acc_v[pl.ds(s_star * 16, 16)] = newacc
            return p_acc

        p_acc = lax.fori_loop(0, M, per_round, jnp.zeros((16,), jnp.int32))
        res_v[pl.ds(cl * 16, 16)] = p_acc

    def pair_body(cp, _):
        col = wid * CPW + 2 * cp
        h_a = pltpu.make_async_copy(a_hbm.at[col], slab_a, sem_a)
        h_b = pltpu.make_async_copy(a_hbm.at[col + 1], slab_b, sem_b)
        h_a.start()
        h_b.start()
        h_a.wait()
        per_column(2 * cp, slab_a)
        h_b.wait()
        per_column(2 * cp + 1, slab_b)
        return _

    lax.fori_loop(0, CPW // 2, pair_body, 0)
    pltpu.sync_copy(res_v, out_hbm.at[pl.ds(wid * (CPW * 16), CPW * 16)])


def kernel(x_input):
    b, length, d = x_input.shape
    x_DFT = jnp.fft.fft(x_input, axis=1)
    a = jnp.abs(x_DFT)
    a_t = jnp.transpose(a, (0, 2, 1)).reshape(b * d, length)
    mesh = plsc.VectorSubcoreMesh(core_axis_name="c", subcore_axis_name="s")
    sc = functools.partial(
        pl.kernel,
        mesh=mesh,
        compiler_params=pltpu.CompilerParams(needs_layout_passes=False),
        out_type=jax.ShapeDtypeStruct((b * d * 16,), jnp.int32),
        scratch_types=[
            pltpu.VMEM((length,), jnp.float32),
            pltpu.VMEM((length,), jnp.float32),
            pltpu.VMEM((NSEG * 16,), jnp.float32),
            pltpu.VMEM((CPW * 16,), jnp.int32),
            pltpu.SemaphoreType.DMA,
            pltpu.SemaphoreType.DMA,
        ],
    )(_sc_select)
    flat = sc(a_t)
    p = jnp.transpose(flat.reshape(b, d, 16)[:, :, :M], (0, 2, 1))
    return p.astype(jnp.int64)
